# baseline (speedup 1.0000x reference)
import jax
import jax.numpy as jnp
from jax.experimental import pallas as pl

B, N = 256, 512
EMB, VIS, KPT = 128, 1, 17
TOKEN_DIM = 128
M = B * N
SMALL = VIS + 4 + KPT * 3

ROWS = 1024


def _body(emb_ref, vis_ref, bbox_ref, kpt_ref, msk_ref, w_emb_ref, w_small_ref,
          bias_ref, out_ref):
    acc = jnp.dot(emb_ref[...], w_emb_ref[...],
                  preferred_element_type=jnp.float32)
    x_small = jnp.concatenate([vis_ref[...], bbox_ref[...], kpt_ref[...]],
                              axis=1)
    acc = acc + jnp.dot(x_small, w_small_ref[...],
                        preferred_element_type=jnp.float32)
    acc = acc + bias_ref[...]
    out_ref[...] = acc * msk_ref[...]


def kernel(feats_masks, embeddings, visibility_scores, bbox_ltwh,
           keypoints_xyc, app_W, app_b, st_W, st_b):
    emb = embeddings.reshape(M, EMB)
    vis = visibility_scores.reshape(M, VIS)
    bbox = bbox_ltwh.reshape(M, 4)
    kpt = keypoints_xyc.reshape(M, KPT * 3)
    msk = feats_masks.reshape(M, 1).astype(jnp.float32)

    w_emb = app_W[:, :EMB].T
    w_small = jnp.concatenate([app_W[:, EMB:], st_W], axis=1).T
    bias = (app_b + st_b).reshape(1, TOKEN_DIM)

    grid = (M // ROWS,)
    row_spec = lambda f: pl.BlockSpec((ROWS, f), lambda i: (i, 0))
    full_spec = lambda s: pl.BlockSpec(s, lambda i: (0, 0))

    out = pl.pallas_call(
        _body,
        grid=grid,
        in_specs=[
            row_spec(EMB),
            row_spec(VIS),
            row_spec(4),
            row_spec(KPT * 3),
            row_spec(1),
            full_spec((EMB, TOKEN_DIM)),
            full_spec((SMALL, TOKEN_DIM)),
            full_spec((1, TOKEN_DIM)),
        ],
        out_specs=pl.BlockSpec((ROWS, TOKEN_DIM), lambda i: (i, 0)),
        out_shape=jax.ShapeDtypeStruct((M, TOKEN_DIM), jnp.float32),
    )(emb, vis, bbox, kpt, msk, w_emb, w_small, bias)
    return out.reshape(B, N, TOKEN_DIM)


# speedup vs baseline: 1.5085x; 1.5085x over previous
import jax
import jax.numpy as jnp
from jax.experimental import pallas as pl

B, N = 256, 512
EMB, VIS, KPT = 128, 1, 17
TOKEN_DIM = 128
M = B * N
SMALL = VIS + 4 + KPT * 3
MASK_COL = SMALL

ROWS = 1024


def _body(emb_ref, xs_ref, w_emb_ref, w_xs_ref, out_ref):
    xs = xs_ref[...]
    acc = jnp.dot(emb_ref[...], w_emb_ref[...],
                  preferred_element_type=jnp.float32)
    acc = acc + jnp.dot(xs, w_xs_ref[...],
                        preferred_element_type=jnp.float32)
    m = jax.lax.slice(xs, (0, MASK_COL), (ROWS, MASK_COL + 1))
    out_ref[...] = acc * m


def kernel(feats_masks, embeddings, visibility_scores, bbox_ltwh,
           keypoints_xyc, app_W, app_b, st_W, st_b):
    emb = embeddings.reshape(M, EMB)

    xs = jnp.concatenate(
        [
            visibility_scores,
            bbox_ltwh,
            keypoints_xyc.reshape(B, N, KPT * 3),
            feats_masks[..., None].astype(jnp.float32),
        ],
        axis=-1,
    )
    xs = jnp.pad(xs, ((0, 0), (0, 0), (0, 128 - (SMALL + 1))))
    xs = xs.reshape(M, 128)

    w_emb = app_W[:, :EMB].T
    w_small = jnp.concatenate([app_W[:, EMB:], st_W], axis=1).T
    bias_row = (app_b + st_b).reshape(1, TOKEN_DIM)
    w_xs = jnp.concatenate(
        [w_small, bias_row, jnp.zeros((128 - SMALL - 1, TOKEN_DIM), jnp.float32)],
        axis=0,
    )

    grid = (M // ROWS,)
    out = pl.pallas_call(
        _body,
        grid=grid,
        in_specs=[
            pl.BlockSpec((ROWS, EMB), lambda i: (i, 0)),
            pl.BlockSpec((ROWS, 128), lambda i: (i, 0)),
            pl.BlockSpec((EMB, TOKEN_DIM), lambda i: (0, 0)),
            pl.BlockSpec((128, TOKEN_DIM), lambda i: (0, 0)),
        ],
        out_specs=pl.BlockSpec((ROWS, TOKEN_DIM), lambda i: (i, 0)),
        out_shape=jax.ShapeDtypeStruct((M, TOKEN_DIM), jnp.float32),
    )(emb, xs, w_emb, w_xs)
    return out.reshape(B, N, TOKEN_DIM)


# speedup vs baseline: 2.0179x; 1.3376x over previous
import jax
import jax.numpy as jnp
from jax.experimental import pallas as pl

B, N = 256, 512
EMB, VIS, KPT = 128, 1, 17
TOKEN_DIM = 128
M = B * N
SMALL = VIS + 4 + KPT * 3
MASK_COL = SMALL

ROWS = 8192


def _body(emb_ref, xs_ref, w_emb_ref, w_xs_ref, out_ref):
    xs = xs_ref[...]
    acc = jnp.dot(emb_ref[...], w_emb_ref[...],
                  preferred_element_type=jnp.float32)
    acc = acc + jnp.dot(xs, w_xs_ref[...],
                        preferred_element_type=jnp.float32)
    m = jax.lax.slice(xs, (0, MASK_COL), (ROWS, MASK_COL + 1))
    out_ref[...] = acc * m


def kernel(feats_masks, embeddings, visibility_scores, bbox_ltwh,
           keypoints_xyc, app_W, app_b, st_W, st_b):
    emb = embeddings.reshape(M, EMB)

    xs = jnp.concatenate(
        [
            visibility_scores,
            bbox_ltwh,
            keypoints_xyc.reshape(B, N, KPT * 3),
            feats_masks[..., None].astype(jnp.float32),
        ],
        axis=-1,
    )
    xs = jnp.pad(xs, ((0, 0), (0, 0), (0, 128 - (SMALL + 1))))
    xs = xs.reshape(M, 128)

    w_emb = app_W[:, :EMB].T
    w_small = jnp.concatenate([app_W[:, EMB:], st_W], axis=1).T
    bias_row = (app_b + st_b).reshape(1, TOKEN_DIM)
    w_xs = jnp.concatenate(
        [w_small, bias_row, jnp.zeros((128 - SMALL - 1, TOKEN_DIM), jnp.float32)],
        axis=0,
    )

    grid = (M // ROWS,)
    out = pl.pallas_call(
        _body,
        grid=grid,
        in_specs=[
            pl.BlockSpec((ROWS, EMB), lambda i: (i, 0)),
            pl.BlockSpec((ROWS, 128), lambda i: (i, 0)),
            pl.BlockSpec((EMB, TOKEN_DIM), lambda i: (0, 0)),
            pl.BlockSpec((128, TOKEN_DIM), lambda i: (0, 0)),
        ],
        out_specs=pl.BlockSpec((ROWS, TOKEN_DIM), lambda i: (i, 0)),
        out_shape=jax.ShapeDtypeStruct((M, TOKEN_DIM), jnp.float32),
    )(emb, xs, w_emb, w_xs)
    return out.reshape(B, N, TOKEN_DIM)


# speedup vs baseline: 2.7070x; 1.3415x over previous
import jax
import jax.numpy as jnp
from jax.experimental import pallas as pl

B, N = 256, 512
EMB, VIS, KPT = 128, 1, 17
TOKEN_DIM = 128
M = B * N
SMALL = VIS + 4 + KPT * 3
MASK_COL = SMALL

ROWS = 8192


def _body(emb_ref, xs_ref, w_emb_ref, w_xs_ref, out_ref):
    xs = xs_ref[...]
    acc = jnp.dot(emb_ref[...], w_emb_ref[...],
                  preferred_element_type=jnp.float32)
    acc = acc + jnp.dot(xs, w_xs_ref[...],
                        preferred_element_type=jnp.float32)
    m = jax.lax.slice(xs, (0, MASK_COL), (ROWS, MASK_COL + 1))
    out_ref[...] = acc * m.astype(jnp.float32)


def kernel(feats_masks, embeddings, visibility_scores, bbox_ltwh,
           keypoints_xyc, app_W, app_b, st_W, st_b):
    emb = embeddings.reshape(M, EMB)

    xs = jnp.concatenate(
        [
            visibility_scores,
            bbox_ltwh,
            keypoints_xyc.reshape(B, N, KPT * 3),
            feats_masks[..., None].astype(jnp.float32),
        ],
        axis=-1,
    ).astype(jnp.bfloat16)
    xs = jnp.pad(xs, ((0, 0), (0, 0), (0, 128 - (SMALL + 1))))
    xs = xs.reshape(M, 128)

    w_emb = app_W[:, :EMB].T
    w_small = jnp.concatenate([app_W[:, EMB:], st_W], axis=1).T
    bias_row = (app_b + st_b).reshape(1, TOKEN_DIM)
    w_xs = jnp.concatenate(
        [w_small, bias_row, jnp.zeros((128 - SMALL - 1, TOKEN_DIM), jnp.float32)],
        axis=0,
    ).astype(jnp.bfloat16)

    grid = (M // ROWS,)
    out = pl.pallas_call(
        _body,
        grid=grid,
        in_specs=[
            pl.BlockSpec((ROWS, EMB), lambda i: (i, 0)),
            pl.BlockSpec((ROWS, 128), lambda i: (i, 0)),
            pl.BlockSpec((EMB, TOKEN_DIM), lambda i: (0, 0)),
            pl.BlockSpec((128, TOKEN_DIM), lambda i: (0, 0)),
        ],
        out_specs=pl.BlockSpec((ROWS, TOKEN_DIM), lambda i: (i, 0)),
        out_shape=jax.ShapeDtypeStruct((M, TOKEN_DIM), jnp.float32),
    )(emb, xs, w_emb, w_xs)
    return out.reshape(B, N, TOKEN_DIM)
